# baseline (device time: 55872 ns/iter reference)
import jax
import jax.numpy as jnp
from jax import lax
from jax.experimental import pallas as pl
from jax.experimental.pallas import tpu as pltpu

K = 32
S = 6
BW = 128
BM = 256

_NEG_INF = float("-inf")


def kernel(x):
    m_rows, n_loc = x.shape
    n_rb = m_rows // BM
    n_ch = n_loc // BW

    def body(x_ref, out_ref, send_ref, recv_ref, send_sem, recv_sem):
        my_x = lax.axis_index("x")
        my_y = lax.axis_index("y")
        my_z = lax.axis_index("z")
        partner = (1 - my_x, my_y, my_z)

        barrier_sem = pltpu.get_barrier_semaphore()
        pl.semaphore_signal(
            barrier_sem,
            inc=1,
            device_id=partner,
            device_id_type=pl.DeviceIdType.MESH,
        )
        pl.semaphore_wait(barrier_sem, 1)

        def row_block(rb, carry):
            rs = pl.ds(rb * BM, BM)
            regs = [jnp.full((BM, BW), _NEG_INF, jnp.float32) for _ in range(S)]
            for c in range(n_ch):
                t = x_ref[rs, pl.ds(c * BW, BW)]
                for s in range(S):
                    hi = jnp.maximum(regs[s], t)
                    t = jnp.minimum(regs[s], t)
                    regs[s] = hi

            frontier = regs[0]
            work = regs[1:]
            for i in range(K):
                m = jnp.max(frontier, axis=1, keepdims=True)
                send_ref[rs, i : i + 1] = m
                hit = frontier == m
                frontier = jnp.where(hit, work[0], frontier)
                for s in range(len(work) - 1):
                    work[s] = jnp.where(hit, work[s + 1], work[s])
                work[-1] = jnp.where(hit, _NEG_INF, work[-1])
            return carry

        lax.fori_loop(0, n_rb, row_block, 0)

        rdma = pltpu.make_async_remote_copy(
            src_ref=send_ref,
            dst_ref=recv_ref,
            send_sem=send_sem,
            recv_sem=recv_sem,
            device_id=partner,
            device_id_type=pl.DeviceIdType.MESH,
        )
        rdma.start()
        rdma.wait()

        def merge_block(rb, carry):
            rs = pl.ds(rb * BM, BM)
            cur = jnp.concatenate([send_ref[rs, :], recv_ref[rs, :]], axis=1)
            m = jnp.max(cur, axis=1, keepdims=True)
            out_ref[rs, 0:1] = m
            for i in range(1, K):
                cur = jnp.where(cur == m, _NEG_INF, cur)
                m = jnp.max(cur, axis=1, keepdims=True)
                out_ref[rs, i : i + 1] = m
            return carry

        lax.fori_loop(0, n_rb, merge_block, 0)

        pl.semaphore_signal(
            barrier_sem,
            inc=1,
            device_id=partner,
            device_id_type=pl.DeviceIdType.MESH,
        )
        pl.semaphore_wait(barrier_sem, 1)

    return pl.pallas_call(
        body,
        out_shape=jax.ShapeDtypeStruct((m_rows, K), jnp.float32),
        in_specs=[pl.BlockSpec(memory_space=pltpu.VMEM)],
        out_specs=pl.BlockSpec(memory_space=pltpu.VMEM),
        scratch_shapes=[
            pltpu.VMEM((m_rows, K), jnp.float32),
            pltpu.VMEM((m_rows, K), jnp.float32),
            pltpu.SemaphoreType.DMA,
            pltpu.SemaphoreType.DMA,
        ],
        compiler_params=pltpu.CompilerParams(
            vmem_limit_bytes=64 * 1024 * 1024,
            collective_id=0,
        ),
    )(x)


# device time: 47384 ns/iter; 1.1791x vs baseline; 1.1791x over previous
import jax
import jax.numpy as jnp
from jax import lax
from jax.experimental import pallas as pl
from jax.experimental.pallas import tpu as pltpu

K = 32
S = 6
BW = 128
BM = 256

_NEG_INF = float("-inf")


def kernel(x):
    m_rows, n_loc = x.shape
    n_rb = m_rows // BM
    n_ch = n_loc // BW

    def body(x_ref, out_ref, xb_ref, send_ref, recv_ref, copy_sems, send_sem, recv_sem):
        my_x = lax.axis_index("x")
        my_y = lax.axis_index("y")
        my_z = lax.axis_index("z")
        partner = (1 - my_x, my_y, my_z)

        def block_copy(rb, slot):
            return pltpu.make_async_copy(
                x_ref.at[pl.ds(rb * BM, BM), :],
                xb_ref.at[slot],
                copy_sems.at[slot],
            )

        block_copy(0, 0).start()

        barrier_sem = pltpu.get_barrier_semaphore()
        pl.semaphore_signal(
            barrier_sem,
            inc=1,
            device_id=partner,
            device_id_type=pl.DeviceIdType.MESH,
        )
        pl.semaphore_wait(barrier_sem, 1)

        def row_block(rb, carry):
            sl = lax.rem(rb, 2)
            block_copy(rb, sl).wait()

            @pl.when(rb + 1 < n_rb)
            def _():
                block_copy(rb + 1, lax.rem(rb + 1, 2)).start()

            rs = pl.ds(rb * BM, BM)
            regs = [jnp.full((BM, BW), _NEG_INF, jnp.float32) for _ in range(S)]
            for c in range(n_ch):
                t = xb_ref[sl, :, pl.ds(c * BW, BW)]
                for s in range(S):
                    hi = jnp.maximum(regs[s], t)
                    t = jnp.minimum(regs[s], t)
                    regs[s] = hi

            frontier = regs[0]
            work = regs[1:]
            for i in range(K):
                m = jnp.max(frontier, axis=1, keepdims=True)
                send_ref[rs, i : i + 1] = m
                hit = frontier == m
                frontier = jnp.where(hit, work[0], frontier)
                for s in range(len(work) - 1):
                    work[s] = jnp.where(hit, work[s + 1], work[s])
                work[-1] = jnp.where(hit, _NEG_INF, work[-1])
            return carry

        lax.fori_loop(0, n_rb, row_block, 0)

        rdma = pltpu.make_async_remote_copy(
            src_ref=send_ref,
            dst_ref=recv_ref,
            send_sem=send_sem,
            recv_sem=recv_sem,
            device_id=partner,
            device_id_type=pl.DeviceIdType.MESH,
        )
        rdma.start()
        rdma.wait()

        def merge_block(rb, carry):
            rs = pl.ds(rb * BM, BM)
            cur = jnp.concatenate([send_ref[rs, :], recv_ref[rs, :]], axis=1)
            m = jnp.max(cur, axis=1, keepdims=True)
            out_ref[rs, 0:1] = m
            for i in range(1, K):
                cur = jnp.where(cur == m, _NEG_INF, cur)
                m = jnp.max(cur, axis=1, keepdims=True)
                out_ref[rs, i : i + 1] = m
            return carry

        lax.fori_loop(0, n_rb, merge_block, 0)

        pl.semaphore_signal(
            barrier_sem,
            inc=1,
            device_id=partner,
            device_id_type=pl.DeviceIdType.MESH,
        )
        pl.semaphore_wait(barrier_sem, 1)

    return pl.pallas_call(
        body,
        out_shape=jax.ShapeDtypeStruct((m_rows, K), jnp.float32),
        in_specs=[pl.BlockSpec(memory_space=pl.ANY)],
        out_specs=pl.BlockSpec(memory_space=pltpu.VMEM),
        scratch_shapes=[
            pltpu.VMEM((2, BM, n_loc), jnp.float32),
            pltpu.VMEM((m_rows, K), jnp.float32),
            pltpu.VMEM((m_rows, K), jnp.float32),
            pltpu.SemaphoreType.DMA((2,)),
            pltpu.SemaphoreType.DMA,
            pltpu.SemaphoreType.DMA,
        ],
        compiler_params=pltpu.CompilerParams(
            vmem_limit_bytes=64 * 1024 * 1024,
            collective_id=0,
        ),
    )(x)
